# baseline (speedup 1.0000x reference)
import functools

import jax
import jax.numpy as jnp
from jax import lax
from jax.experimental import pallas as pl
from jax.experimental.pallas import tpu as pltpu
from jax.experimental.pallas import tpu_sc as plsc

N = 10000
E = 320000
G = 512
NC = 2
NS = 16
CH = 128
IB = 8
NCH0 = 80
NCH1 = 160
RPT = 632
NROWS = NS * RPT


def _sc_agg_body(nchunk, feat_split, xs_hbm, idx_hbm, zer_hbm,
                 out_hbm, idx_v, gbuf, agg_sp, sem):
    c = lax.axis_index("c")
    s = lax.axis_index("s")
    pltpu.sync_copy(zer_hbm.at[pl.ds(s * RPT, RPT)],
                    agg_sp.at[pl.ds(s * RPT, RPT)])
    w = c * NS + s if not feat_split else s
    plsc.subcore_barrier()

    x_src = xs_hbm.at[c] if feat_split else xs_hbm
    my_idx = idx_hbm.at[w]

    def gstart(ib, i, gslot, gsem):
        pltpu.async_copy(x_src.at[ib.at[i].at[0]], gslot, gsem)

    def gwait(ib, i, gslot, gsem):
        pltpu.make_async_copy(x_src.at[ib.at[i].at[0]], gslot, gsem).wait()

    def sstart(ib, i, gslot, tsem):
        pltpu.async_copy(gslot, agg_sp.at[ib.at[i].at[1]], tsem, add=True)

    def swait(ib, i, gslot, tsem):
        pltpu.make_async_copy(gslot, agg_sp.at[ib.at[i].at[1]], tsem).wait()

    ia, ib_ = idx_v.at[0], idx_v.at[1]
    gs = (gbuf.at[0], gbuf.at[1])
    ss = (sem.at[0], sem.at[1])
    ts = (sem.at[2], sem.at[3])
    nblk = nchunk // IB

    def run_block(iblk, prev, inext_j, inext, is_last):
        pv, pi = prev
        for i in range(IB):
            p, q = i % 2, 1 - i % 2
            gwait(iblk, i, gs[p], ss[p])
            sstart(iblk, i, gs[p], ts[p])
            if i == 0:
                swait(pv, pi, gs[q], ts[q])
                gstart(iblk, 1, gs[q], ss[q])
            elif i < IB - 1:
                swait(iblk, i - 1, gs[q], ts[q])
                gstart(iblk, i + 1, gs[q], ss[q])
            elif is_last is None:
                pltpu.sync_copy(my_idx.at[inext_j], inext)
                swait(iblk, IB - 2, gs[q], ts[q])
                gstart(inext, 0, gs[q], ss[q])
            else:
                @pl.when(~is_last)
                def _():
                    pltpu.sync_copy(my_idx.at[inext_j], inext)
                    swait(iblk, IB - 2, gs[q], ts[q])
                    gstart(inext, 0, gs[q], ss[q])

                @pl.when(is_last)
                def _():
                    swait(iblk, IB - 2, gs[q], ts[q])
                    swait(iblk, IB - 1, gs[p], ts[p])

    pltpu.sync_copy(zer_hbm.at[pl.ds(s * RPT, CH)], gs[1])
    pltpu.sync_copy(my_idx.at[0], ia)
    pltpu.sync_copy(my_idx.at[1], ib_)
    gstart(ia, 0, gs[0], ss[0])
    sstart(ib_, IB - 1, gs[1], ts[1])

    def pair(bp, carry):
        b = 2 * bp
        run_block(ia, (ib_, IB - 1), b + 1, ib_, None)
        run_block(ib_, (ia, IB - 1), b + 2, ia, b + 2 >= nblk)
        return carry

    lax.fori_loop(0, nblk // 2, pair, 0)
    plsc.subcore_barrier()

    @pl.when(s < NS - 1)
    def _():
        pltpu.sync_copy(agg_sp.at[pl.ds(s * RPT, RPT)],
                        out_hbm.at[c].at[pl.ds(s * RPT, RPT)])

    @pl.when(s == NS - 1)
    def _():
        last = N - (NS - 1) * RPT
        pltpu.sync_copy(agg_sp.at[pl.ds((NS - 1) * RPT, last)],
                        out_hbm.at[c].at[pl.ds((NS - 1) * RPT, last)])


def _make_sc_agg(nchunk, feat_split):
    mesh = plsc.VectorSubcoreMesh(core_axis_name="c", subcore_axis_name="s",
                                  num_cores=NC, num_subcores=NS)
    return pl.kernel(
        functools.partial(_sc_agg_body, nchunk, feat_split),
        out_type=jax.ShapeDtypeStruct((NC, N, 128), jnp.float32),
        mesh=mesh,
        scratch_types=[
            pltpu.VMEM((2, IB, 2, CH), jnp.int32),
            pltpu.VMEM((2, CH, 128), jnp.float32),
            pltpu.VMEM_SHARED((NROWS, 128), jnp.float32),
            pltpu.SemaphoreType.DMA((4,)),
        ],
    )


_EPS = 1e-5


def _mlp0_body(x_ref, agg_ref, W1, b1, g1, be1, W2, b2, bg, bb, o_ref):
    h = x_ref[...] + agg_ref[0] + agg_ref[1]
    z = jnp.dot(h, W1[...], preferred_element_type=jnp.float32) + b1[...]
    m = jnp.mean(z, axis=0)
    v = jnp.mean((z - m) ** 2, axis=0)
    z = jnp.maximum(g1[...] * (z - m) / jnp.sqrt(v + _EPS) + be1[...], 0.0)
    z = jnp.maximum(jnp.dot(z, W2[...], preferred_element_type=jnp.float32)
                    + b2[...], 0.0)
    m2 = jnp.mean(z, axis=0)
    v2 = jnp.mean((z - m2) ** 2, axis=0)
    z = jnp.maximum(bg[...] * (z - m2) / jnp.sqrt(v2 + _EPS) + bb[...], 0.0)
    o_ref[0] = z[:, :128]
    o_ref[1] = z[:, 128:]


def _mlp1_body(h_ref, agg_ref, batch_ref, W1, b1, g1, be1, W2, b2, bg, bb,
               fcW, fcb, o_ref):
    h = (jnp.concatenate([h_ref[0], h_ref[1]], axis=1)
         + jnp.concatenate([agg_ref[0], agg_ref[1]], axis=1))
    z = jnp.dot(h, W1[...], preferred_element_type=jnp.float32) + b1[...]
    m = jnp.mean(z, axis=0)
    v = jnp.mean((z - m) ** 2, axis=0)
    z = jnp.maximum(g1[...] * (z - m) / jnp.sqrt(v + _EPS) + be1[...], 0.0)
    z = jnp.maximum(jnp.dot(z, W2[...], preferred_element_type=jnp.float32)
                    + b2[...], 0.0)
    m2 = jnp.mean(z, axis=0)
    v2 = jnp.mean((z - m2) ** 2, axis=0)
    z = jnp.maximum(bg[...] * (z - m2) / jnp.sqrt(v2 + _EPS) + bb[...], 0.0)
    oh = (lax.broadcasted_iota(jnp.int32, (G, N), 0)
          == batch_ref[...][None, :]).astype(jnp.float32)
    pooled = jnp.dot(oh, z, preferred_element_type=jnp.float32)
    o = jnp.dot(pooled, fcW[...], preferred_element_type=jnp.float32) + fcb[...]
    o = o - jnp.max(o, axis=1, keepdims=True)
    o_ref[...] = o - jnp.log(jnp.sum(jnp.exp(o), axis=1, keepdims=True))


def kernel(x, edge_index, batch,
           W1_0, b1_0, g1_0, be1_0, W2_0, b2_0, bn0_g, bn0_b,
           W1_1, b1_1, g1_1, be1_1, W2_1, b2_1, bn1_g, bn1_b,
           fc_W, fc_b):
    src = edge_index[0]
    dst = edge_index[1]
    def pack(nworker, nchunk):
        pad = nworker * nchunk * CH - E
        trash = N + jnp.arange(pad, dtype=jnp.int32) % (NROWS - N)
        psrc = jnp.arange(pad, dtype=jnp.int32) % N
        s = jnp.concatenate([src, psrc]).reshape(
            nworker, nchunk // IB, IB, 1, CH)
        d = jnp.concatenate([dst, trash]).reshape(
            nworker, nchunk // IB, IB, 1, CH)
        return jnp.concatenate([s, d], axis=3)

    idx0 = pack(NC * NS, NCH0)
    idx1 = pack(NS, NCH1)

    zer = jnp.zeros((NROWS, 128), jnp.float32)

    agg0 = _make_sc_agg(NCH0, False)(x, idx0, zer)

    h1s = pl.pallas_call(
        _mlp0_body,
        out_shape=jax.ShapeDtypeStruct((NC, N, 128), jnp.float32),
    )(x, agg0, W1_0, b1_0, g1_0, be1_0, W2_0, b2_0, bn0_g, bn0_b)

    agg1 = _make_sc_agg(NCH1, True)(h1s, idx1, zer)

    out = pl.pallas_call(
        _mlp1_body,
        out_shape=jax.ShapeDtypeStruct((G, 64), jnp.float32),
    )(h1s, agg1, batch, W1_1, b1_1, g1_1, be1_1, W2_1, b2_1,
      bn1_g, bn1_b, fc_W, fc_b)
    return out


# speedup vs baseline: 1.2003x; 1.2003x over previous
import functools

import jax
import jax.numpy as jnp
from jax import lax
from jax.experimental import pallas as pl
from jax.experimental.pallas import tpu as pltpu
from jax.experimental.pallas import tpu_sc as plsc

N = 10000
E = 320000
G = 512
NC = 2
NS = 16
CH = 128
IB = 8
NCH0 = 80
NCH1 = 160
RPT = 632
NROWS = NS * RPT


def _sc_agg_body(nchunk, feat_split, xs_hbm, idx_hbm, zer_hbm,
                 out_hbm, idx_v, gbuf, agg_sp, sem):
    c = lax.axis_index("c")
    s = lax.axis_index("s")
    pltpu.sync_copy(zer_hbm.at[pl.ds(s * RPT, RPT)],
                    agg_sp.at[pl.ds(s * RPT, RPT)])
    w = c * NS + s if not feat_split else s
    plsc.subcore_barrier()

    x_src = xs_hbm.at[c] if feat_split else xs_hbm
    my_idx = idx_hbm.at[w]

    def gstart(ib, i, gslot, gsem):
        pltpu.async_copy(x_src.at[ib.at[i].at[0]], gslot, gsem)

    def gdrain(ib, i, gslot, gsem):
        pltpu.make_async_copy(x_src.at[ib.at[i].at[0]], gslot, gsem).wait()
        pltpu.sync_copy(gslot, agg_sp.at[ib.at[i].at[1]], add=True)

    ia, ib_ = idx_v.at[0], idx_v.at[1]
    g0, g1 = gbuf.at[0], gbuf.at[1]
    s0, s1 = sem.at[0], sem.at[1]
    gs = (g0, g1)
    ss = (s0, s1)
    nblk = nchunk // IB

    def run_block(iblk, inext_j, inext, is_last):
        for i in range(IB - 2):
            gdrain(iblk, i, gs[i % 2], ss[i % 2])
            gstart(iblk, i + 2, gs[i % 2], ss[i % 2])
        if is_last is None:
            pltpu.sync_copy(my_idx.at[inext_j], inext)
            gdrain(iblk, IB - 2, g0, s0)
            gstart(inext, 0, g0, s0)
            gdrain(iblk, IB - 1, g1, s1)
            gstart(inext, 1, g1, s1)
        else:
            @pl.when(~is_last)
            def _():
                pltpu.sync_copy(my_idx.at[inext_j], inext)
                gdrain(iblk, IB - 2, g0, s0)
                gstart(inext, 0, g0, s0)
                gdrain(iblk, IB - 1, g1, s1)
                gstart(inext, 1, g1, s1)

            @pl.when(is_last)
            def _():
                gdrain(iblk, IB - 2, g0, s0)
                gdrain(iblk, IB - 1, g1, s1)

    pltpu.sync_copy(my_idx.at[0], ia)
    gstart(ia, 0, g0, s0)
    gstart(ia, 1, g1, s1)

    def pair(bp, carry):
        b = 2 * bp
        run_block(ia, b + 1, ib_, None)
        run_block(ib_, b + 2, ia, b + 2 >= nblk)
        return carry

    lax.fori_loop(0, nblk // 2, pair, 0)
    plsc.subcore_barrier()

    @pl.when(s < NS - 1)
    def _():
        pltpu.sync_copy(agg_sp.at[pl.ds(s * RPT, RPT)],
                        out_hbm.at[c].at[pl.ds(s * RPT, RPT)])

    @pl.when(s == NS - 1)
    def _():
        last = N - (NS - 1) * RPT
        pltpu.sync_copy(agg_sp.at[pl.ds((NS - 1) * RPT, last)],
                        out_hbm.at[c].at[pl.ds((NS - 1) * RPT, last)])


def _make_sc_agg(nchunk, feat_split):
    mesh = plsc.VectorSubcoreMesh(core_axis_name="c", subcore_axis_name="s",
                                  num_cores=NC, num_subcores=NS)
    return pl.kernel(
        functools.partial(_sc_agg_body, nchunk, feat_split),
        out_type=jax.ShapeDtypeStruct((NC, N, 128), jnp.float32),
        mesh=mesh,
        scratch_types=[
            pltpu.VMEM((2, IB, 2, CH), jnp.int32),
            pltpu.VMEM((2, CH, 128), jnp.float32),
            pltpu.VMEM_SHARED((NROWS, 128), jnp.float32),
            pltpu.SemaphoreType.DMA((2,)),
        ],
    )


_EPS = 1e-5


def _mlp0_body(x_ref, agg_ref, W1, b1, g1, be1, W2, b2, bg, bb, o_ref):
    h = x_ref[...] + agg_ref[0] + agg_ref[1]
    z = jnp.dot(h, W1[...], preferred_element_type=jnp.float32) + b1[...]
    m = jnp.mean(z, axis=0)
    v = jnp.mean((z - m) ** 2, axis=0)
    z = jnp.maximum(g1[...] * (z - m) / jnp.sqrt(v + _EPS) + be1[...], 0.0)
    z = jnp.maximum(jnp.dot(z, W2[...], preferred_element_type=jnp.float32)
                    + b2[...], 0.0)
    m2 = jnp.mean(z, axis=0)
    v2 = jnp.mean((z - m2) ** 2, axis=0)
    z = jnp.maximum(bg[...] * (z - m2) / jnp.sqrt(v2 + _EPS) + bb[...], 0.0)
    o_ref[0] = z[:, :128]
    o_ref[1] = z[:, 128:]


def _mlp1_body(h_ref, agg_ref, batch_ref, W1, b1, g1, be1, W2, b2, bg, bb,
               fcW, fcb, o_ref):
    h = (jnp.concatenate([h_ref[0], h_ref[1]], axis=1)
         + jnp.concatenate([agg_ref[0], agg_ref[1]], axis=1))
    z = jnp.dot(h, W1[...], preferred_element_type=jnp.float32) + b1[...]
    m = jnp.mean(z, axis=0)
    v = jnp.mean((z - m) ** 2, axis=0)
    z = jnp.maximum(g1[...] * (z - m) / jnp.sqrt(v + _EPS) + be1[...], 0.0)
    z = jnp.maximum(jnp.dot(z, W2[...], preferred_element_type=jnp.float32)
                    + b2[...], 0.0)
    m2 = jnp.mean(z, axis=0)
    v2 = jnp.mean((z - m2) ** 2, axis=0)
    z = jnp.maximum(bg[...] * (z - m2) / jnp.sqrt(v2 + _EPS) + bb[...], 0.0)
    oh = (lax.broadcasted_iota(jnp.int32, (G, N), 0)
          == batch_ref[...][None, :]).astype(jnp.float32)
    pooled = jnp.dot(oh, z, preferred_element_type=jnp.float32)
    o = jnp.dot(pooled, fcW[...], preferred_element_type=jnp.float32) + fcb[...]
    o = o - jnp.max(o, axis=1, keepdims=True)
    o_ref[...] = o - jnp.log(jnp.sum(jnp.exp(o), axis=1, keepdims=True))


def kernel(x, edge_index, batch,
           W1_0, b1_0, g1_0, be1_0, W2_0, b2_0, bn0_g, bn0_b,
           W1_1, b1_1, g1_1, be1_1, W2_1, b2_1, bn1_g, bn1_b,
           fc_W, fc_b):
    src = edge_index[0]
    dst = edge_index[1]
    def pack(nworker, nchunk):
        pad = nworker * nchunk * CH - E
        trash = N + jnp.arange(pad, dtype=jnp.int32) % (NROWS - N)
        psrc = jnp.arange(pad, dtype=jnp.int32) % N
        s = jnp.concatenate([src, psrc]).reshape(
            nworker, nchunk // IB, IB, 1, CH)
        d = jnp.concatenate([dst, trash]).reshape(
            nworker, nchunk // IB, IB, 1, CH)
        return jnp.concatenate([s, d], axis=3)

    idx0 = pack(NC * NS, NCH0)
    idx1 = pack(NS, NCH1)

    zer = jnp.zeros((NROWS, 128), jnp.float32)

    agg0 = _make_sc_agg(NCH0, False)(x, idx0, zer)

    h1s = pl.pallas_call(
        _mlp0_body,
        out_shape=jax.ShapeDtypeStruct((NC, N, 128), jnp.float32),
    )(x, agg0, W1_0, b1_0, g1_0, be1_0, W2_0, b2_0, bn0_g, bn0_b)

    agg1 = _make_sc_agg(NCH1, True)(h1s, idx1, zer)

    out = pl.pallas_call(
        _mlp1_body,
        out_shape=jax.ShapeDtypeStruct((G, 64), jnp.float32),
    )(h1s, agg1, batch, W1_1, b1_1, g1_1, be1_1, W2_1, b2_1,
      bn1_g, bn1_b, fc_W, fc_b)
    return out
